# baseline (speedup 1.0000x reference)
import functools

import jax
import jax.numpy as jnp
from jax import lax
from jax.experimental import pallas as pl
from jax.experimental.pallas import tpu as pltpu
from jax.experimental.pallas import tpu_sc as plsc

_NUM_SEGMENTS = 100000
_LANE = 128
_NC = 2
_NS = 16
_NW = _NC * _NS
_QSTEPS = 128
_CROWS = _QSTEPS * 16 // _LANE

_SP = ((_NUM_SEGMENTS + _NS * _LANE - 1) // (_NS * _LANE)) * (_NS * _LANE)
_STRIPE = _SP // _NS
_NDUMMY = ((_SP - _NUM_SEGMENTS) // 16) * 16


def _make_sc_accumulate(stripe_elems, nchunks):
    mesh = plsc.VectorSubcoreMesh(core_axis_name="c", subcore_axis_name="s")
    chunk_elems = _QSTEPS * 16

    @functools.partial(
        pl.kernel,
        mesh=mesh,
        out_type=(
            jax.ShapeDtypeStruct((_NC, _SP), jnp.float32),
            jax.ShapeDtypeStruct((_NC, _SP), jnp.float32),
        ),
        scratch_types=[
            pltpu.VMEM((2, 1, chunk_elems), jnp.float32),
            pltpu.VMEM((2, 1, chunk_elems + 16), jnp.int32),
            pltpu.VMEM((2, _CROWS // 8, 8, _LANE), jnp.int32),
            pltpu.VMEM((8, _QSTEPS // 8, 1, 16), jnp.int32),
            pltpu.VMEM((8, _QSTEPS // 8, 1, 16), jnp.float32),
            pltpu.VMEM((16,), jnp.float32),
            pltpu.VMEM((_STRIPE,), jnp.float32),
            pltpu.VMEM_SHARED((_SP,), jnp.float32),
            pltpu.VMEM_SHARED((_SP,), jnp.float32),
            pltpu.SemaphoreType.DMA,
            pltpu.SemaphoreType.DMA,
            pltpu.SemaphoreType.DMA,
            pltpu.SemaphoreType.DMA,
        ],
    )
    def sc_k(e_hbm, id_hbm, idr_hbm, psum_hbm, pcnt_hbm,
             et, idt, idr, cidx, cval, cntbuf, zbuf, sums_sh, cnts_sh,
             sem_l0, sem_l1, sem_s, sem_c):
        c = lax.axis_index("c")
        s = lax.axis_index("s")
        wid = s * _NC + c
        base_elem = wid * stripe_elems
        base_row = base_elem // _LANE

        zero16f = jnp.zeros((16,), jnp.float32)
        one16f = zero16f + 1.0
        vio = lax.iota(jnp.int32, 16)
        fold_ix = [jnp.bitwise_xor(vio, w) for w in (8, 4, 2, 1)]

        gdn = lax.GatherDimensionNumbers(
            offset_dims=(), collapsed_slice_dims=(0,), start_index_map=(0,))

        def any_nonzero(d):
            for ix in fold_ix:
                s = lax.gather(d, ix[:, None], gdn, slice_sizes=(1,),
                               mode=lax.GatherScatterMode.PROMISE_IN_BOUNDS)
                d = d | s
            return d[0] != 0

        def zfill(i, carry):
            zbuf[pl.ds(i * 16, 16)] = zero16f
            return carry

        lax.fori_loop(0, _STRIPE // 16, zfill, 0)
        pltpu.sync_copy(zbuf, sums_sh.at[pl.ds(s * _STRIPE, _STRIPE)])
        pltpu.sync_copy(zbuf, cnts_sh.at[pl.ds(s * _STRIPE, _STRIPE)])
        plsc.subcore_barrier()

        sem_l = (sem_l0, sem_l1)

        def start_loads(ch, b):
            o = base_elem + ch * chunk_elems
            r8 = base_row // 8 + ch * (_CROWS // 8)
            pltpu.async_copy(e_hbm.at[pl.ds(o, chunk_elems)], et.at[b, 0],
                             sem_l[b])
            pltpu.async_copy(id_hbm.at[pl.ds(o, chunk_elems + 16)],
                             idt.at[b, 0], sem_l[b])
            pltpu.async_copy(idr_hbm.at[pl.ds(r8, _CROWS // 8)], idr.at[b],
                             sem_l[b])

        def wait_loads(b):
            pltpu.make_async_copy(
                e_hbm.at[pl.ds(0, chunk_elems)], et.at[b, 0], sem_l[b]).wait()
            pltpu.make_async_copy(
                id_hbm.at[pl.ds(0, chunk_elems + 16)], idt.at[b, 0],
                sem_l[b]).wait()
            pltpu.make_async_copy(
                idr_hbm.at[pl.ds(0, _CROWS // 8)], idr.at[b], sem_l[b]).wait()

        start_loads(0, 0)
        start_loads(1, 1)
        cntbuf[pl.ds(0, 16)] = zero16f

        def pairbody(k, carry):
            for b in range(2):
                ch = k * 2 + b
                wait_loads(b)

                def firerow(jj, carry2):
                    for j8 in range(8):
                        o = (jj * 8 + j8) * _LANE
                        pltpu.async_copy(
                            et.at[b, 0, pl.ds(o, _LANE)],
                            sums_sh.at[idr.at[b, jj, j8]], sem_s, add=True)
                    return carry2

                lax.fori_loop(0, _CROWS // 8, firerow, 0)

                def stepgrp(g, nfired):
                    cnt2 = cntbuf[pl.ds(0, 16)]
                    for u in range(8):
                        q = g * 8 + u
                        qo = q * 16
                        v_id = idt[b, 0, pl.ds(qo, 16)]
                        v_nid = idt[b, 0, pl.ds(qo + 16, 16)]
                        cnt2 = cnt2 + one16f
                        m = v_id != v_nid
                        emit = any_nonzero(v_nid - v_id)
                        dummy = (_NUM_SEGMENTS
                                 + (q * 16) % _NDUMMY) + vio
                        cidx[u, g, 0] = jnp.where(m, v_id, dummy)
                        cval[u, g, 0] = jnp.where(m, cnt2, zero16f)
                        cnt2 = jnp.where(m, zero16f, cnt2)

                        @pl.when(emit)
                        def _():
                            pltpu.async_copy(
                                cval.at[u, g, 0], cnts_sh.at[cidx.at[u, g, 0]],
                                sem_c, add=True)

                        nfired = nfired + jnp.where(emit, 1, 0)
                    cntbuf[pl.ds(0, 16)] = cnt2
                    return nfired

                nfired = lax.fori_loop(0, _QSTEPS // 8, stepgrp,
                                       jnp.int32(0))

                def drain(i, carry2):
                    pltpu.make_async_copy(
                        cval.at[0, 0, 0], cnts_sh.at[cidx.at[0, 0, 0]],
                        sem_c).wait()
                    return carry2

                lax.fori_loop(0, nfired, drain, 0)

                def drainrow(jj, carry2):
                    for j8 in range(8):
                        pltpu.make_async_copy(
                            et.at[b, 0, pl.ds(0, _LANE)],
                            sums_sh.at[idr.at[b, jj, j8]], sem_s).wait()
                    return carry2

                lax.fori_loop(0, _CROWS // 8, drainrow, 0)

                @pl.when(ch + 2 < nchunks)
                def _():
                    start_loads(ch + 2, b)
            return carry

        lax.fori_loop(0, nchunks // 2, pairbody, 0)

        blast = (nchunks - 1) % 2
        v_idlast = idt[blast, 0, pl.ds((_QSTEPS - 1) * 16, 16)]
        cidx[0, 0, 0] = v_idlast
        cval[0, 0, 0] = cntbuf[pl.ds(0, 16)]
        pltpu.sync_copy(cval.at[0, 0, 0], cnts_sh.at[cidx.at[0, 0, 0]],
                        add=True)

        plsc.subcore_barrier()
        sl = pl.ds(s * _STRIPE, _STRIPE)
        pltpu.sync_copy(sums_sh.at[sl], psum_hbm.at[c, sl])
        pltpu.sync_copy(cnts_sh.at[sl], pcnt_hbm.at[c, sl])

    return sc_k


def _tc_finalize(ps_ref, pc_ref, o_ref):
    total = ps_ref[0] + ps_ref[1]
    count = pc_ref[0] + pc_ref[1]
    o_ref[...] = total / jnp.maximum(count, 1.0)


def kernel(site_energy, segment_ids, num_crystals):
    n = site_energy.shape[0]
    flat = site_energy.reshape(n)

    block = _NW * 16 * _QSTEPS
    n_pad = ((n + block - 1) // block) * block
    pad = n_pad - n
    flat = jnp.pad(flat, (0, pad))
    ids = jnp.pad(segment_ids, (0, pad), constant_values=_NUM_SEGMENTS)

    stripe_elems = n_pad // _NW
    sub = stripe_elems // 16
    nchunks = sub // _QSTEPS

    def tr(x):
        return x.reshape(_NW, 16, sub).transpose(0, 2, 1).reshape(-1)

    e_t = tr(flat)
    id_t = tr(ids)
    id_pad = jnp.pad(id_t, (0, 16), constant_values=_NUM_SEGMENTS)
    id_rows = id_t.reshape(n_pad // (8 * _LANE), 8, _LANE)

    psum, pcnt = _make_sc_accumulate(stripe_elems, nchunks)(
        e_t, id_pad, id_rows)

    srows = _SP // _LANE
    mean2d = pl.pallas_call(
        _tc_finalize,
        out_shape=jax.ShapeDtypeStruct((srows, _LANE), jnp.float32),
    )(psum.reshape(_NC, srows, _LANE), pcnt.reshape(_NC, srows, _LANE))

    return mean2d.reshape(_SP)[:_NUM_SEGMENTS, None]


# speedup vs baseline: 1.8700x; 1.8700x over previous
import functools

import jax
import jax.numpy as jnp
from jax import lax
from jax.experimental import pallas as pl
from jax.experimental.pallas import tpu as pltpu
from jax.experimental.pallas import tpu_sc as plsc

_NUM_SEGMENTS = 100000
_LANE = 128
_NC = 2
_NS = 16
_NW = _NC * _NS
_KROWS = 8

_SP = ((_NUM_SEGMENTS + _NS * _LANE - 1) // (_NS * _LANE)) * (_NS * _LANE)
_STRIPE = _SP // _NS


def _make_sc_accumulate(rows_per_worker, nchunks):
    mesh = plsc.VectorSubcoreMesh(core_axis_name="c", subcore_axis_name="s")

    @functools.partial(
        pl.kernel,
        mesh=mesh,
        out_type=(
            jax.ShapeDtypeStruct((_NC, _SP), jnp.float32),
            jax.ShapeDtypeStruct((_NC, _SP), jnp.float32),
        ),
        scratch_types=[
            pltpu.VMEM((2, _KROWS, _LANE), jnp.float32),
            pltpu.VMEM((2, _KROWS, _LANE), jnp.int32),
            pltpu.VMEM((_LANE,), jnp.float32),
            pltpu.VMEM((_STRIPE,), jnp.float32),
            pltpu.VMEM_SHARED((_SP,), jnp.float32),
            pltpu.VMEM_SHARED((_SP,), jnp.float32),
            pltpu.SemaphoreType.DMA,
            pltpu.SemaphoreType.DMA,
            pltpu.SemaphoreType.DMA,
        ],
    )
    def sc_k(e_hbm, id_hbm, psum_hbm, pcnt_hbm,
             ebuf, idbuf, ones, zbuf, sums_sh, cnts_sh,
             sem_l0, sem_l1, sem_s):
        c = lax.axis_index("c")
        s = lax.axis_index("s")
        wid = s * _NC + c

        zero16 = jnp.zeros((16,), jnp.float32)
        for i in range(_LANE // 16):
            ones[pl.ds(i * 16, 16)] = zero16 + 1.0

        def zfill(i, carry):
            zbuf[pl.ds(i * 16, 16)] = zero16
            return carry

        lax.fori_loop(0, _STRIPE // 16, zfill, 0)
        pltpu.sync_copy(zbuf, sums_sh.at[pl.ds(s * _STRIPE, _STRIPE)])
        pltpu.sync_copy(zbuf, cnts_sh.at[pl.ds(s * _STRIPE, _STRIPE)])
        plsc.subcore_barrier()

        base = wid * rows_per_worker
        sem_l = (sem_l0, sem_l1)

        def start_loads(i, b):
            r0 = base + i * _KROWS
            pltpu.async_copy(e_hbm.at[pl.ds(r0, _KROWS)], ebuf.at[b], sem_l[b])
            pltpu.async_copy(id_hbm.at[pl.ds(r0, _KROWS)], idbuf.at[b], sem_l[b])

        def wait_loads(b):
            pltpu.make_async_copy(
                e_hbm.at[pl.ds(0, _KROWS)], ebuf.at[b], sem_l[b]).wait()
            pltpu.make_async_copy(
                id_hbm.at[pl.ds(0, _KROWS)], idbuf.at[b], sem_l[b]).wait()

        start_loads(0, 0)
        start_loads(1, 1)

        def pair(k, carry):
            for b in range(2):
                i = k * 2 + b
                wait_loads(b)
                descs = []
                for j in range(_KROWS):
                    descs.append(pltpu.async_copy(
                        ebuf.at[b, j], sums_sh.at[idbuf.at[b, j]], sem_s,
                        add=True))
                    descs.append(pltpu.async_copy(
                        ones, cnts_sh.at[idbuf.at[b, j]], sem_s, add=True))
                for d in descs:
                    d.wait()

                @pl.when(i + 2 < nchunks)
                def _():
                    start_loads(i + 2, b)
            return carry

        lax.fori_loop(0, nchunks // 2, pair, 0)
        plsc.subcore_barrier()

        sl = pl.ds(s * _STRIPE, _STRIPE)
        pltpu.sync_copy(sums_sh.at[sl], psum_hbm.at[c, sl])
        pltpu.sync_copy(cnts_sh.at[sl], pcnt_hbm.at[c, sl])

    return sc_k


def _tc_finalize(ps_ref, pc_ref, o_ref):
    total = ps_ref[0] + ps_ref[1]
    count = pc_ref[0] + pc_ref[1]
    o_ref[...] = total / jnp.maximum(count, 1.0)


def kernel(site_energy, segment_ids, num_crystals):
    n = site_energy.shape[0]
    flat = site_energy.reshape(n)

    block = _NW * _KROWS * _LANE
    n_pad = ((n + block - 1) // block) * block
    pad = n_pad - n
    flat = jnp.pad(flat, (0, pad))
    ids = jnp.pad(segment_ids, (0, pad), constant_values=_NUM_SEGMENTS)

    rows = n_pad // _LANE
    rows_per_worker = rows // _NW
    nchunks = rows_per_worker // _KROWS

    e2d = flat.reshape(rows, _LANE)
    id2d = ids.reshape(rows, _LANE)

    psum, pcnt = _make_sc_accumulate(rows_per_worker, nchunks)(e2d, id2d)

    srows = _SP // _LANE
    mean2d = pl.pallas_call(
        _tc_finalize,
        out_shape=jax.ShapeDtypeStruct((srows, _LANE), jnp.float32),
    )(psum.reshape(_NC, srows, _LANE), pcnt.reshape(_NC, srows, _LANE))

    return mean2d.reshape(_SP)[:_NUM_SEGMENTS, None]


# speedup vs baseline: 2.1993x; 1.1761x over previous
import functools

import jax
import jax.numpy as jnp
from jax import lax
from jax.experimental import pallas as pl
from jax.experimental.pallas import tpu as pltpu
from jax.experimental.pallas import tpu_sc as plsc

_NUM_SEGMENTS = 100000
_LANE = 128
_NC = 2
_NS = 16
_NW = _NC * _NS
_KROWS = 8

_SP = ((_NUM_SEGMENTS + _NS * _LANE - 1) // (_NS * _LANE)) * (_NS * _LANE)
_STRIPE = _SP // _NS


def _make_sc_accumulate(rows_per_worker, nchunks):
    mesh = plsc.VectorSubcoreMesh(core_axis_name="c", subcore_axis_name="s")

    @functools.partial(
        pl.kernel,
        mesh=mesh,
        out_type=(
            jax.ShapeDtypeStruct((_NC, _SP), jnp.float32),
            jax.ShapeDtypeStruct((_NC, _SP), jnp.float32),
        ),
        scratch_types=[
            pltpu.VMEM((3, _KROWS, _LANE), jnp.float32),
            pltpu.VMEM((3, _KROWS, _LANE), jnp.int32),
            pltpu.VMEM((_LANE,), jnp.float32),
            pltpu.VMEM((_STRIPE,), jnp.float32),
            pltpu.VMEM_SHARED((_SP,), jnp.float32),
            pltpu.VMEM_SHARED((_SP,), jnp.float32),
            pltpu.SemaphoreType.DMA,
            pltpu.SemaphoreType.DMA,
            pltpu.SemaphoreType.DMA,
            pltpu.SemaphoreType.DMA,
            pltpu.SemaphoreType.DMA,
            pltpu.SemaphoreType.DMA,
        ],
    )
    def sc_k(e_hbm, id_hbm, psum_hbm, pcnt_hbm,
             ebuf, idbuf, ones, zbuf, sums_sh, cnts_sh,
             sem_l0, sem_l1, sem_l2, sem_s0, sem_s1, sem_s2):
        c = lax.axis_index("c")
        s = lax.axis_index("s")
        wid = s * _NC + c

        zero16 = jnp.zeros((16,), jnp.float32)
        for i in range(_LANE // 16):
            ones[pl.ds(i * 16, 16)] = zero16 + 1.0

        def zfill(i, carry):
            zbuf[pl.ds(i * 16, 16)] = zero16
            return carry

        lax.fori_loop(0, _STRIPE // 16, zfill, 0)
        pltpu.sync_copy(zbuf, sums_sh.at[pl.ds(s * _STRIPE, _STRIPE)])
        pltpu.sync_copy(zbuf, cnts_sh.at[pl.ds(s * _STRIPE, _STRIPE)])
        plsc.subcore_barrier()

        base = wid * rows_per_worker
        sem_l = (sem_l0, sem_l1, sem_l2)
        sem_s = (sem_s0, sem_s1, sem_s2)

        def start_loads(i, b):
            r0 = base + i * _KROWS
            pltpu.async_copy(e_hbm.at[pl.ds(r0, _KROWS)], ebuf.at[b], sem_l[b])
            pltpu.async_copy(id_hbm.at[pl.ds(r0, _KROWS)], idbuf.at[b], sem_l[b])

        def wait_loads(b):
            pltpu.make_async_copy(
                e_hbm.at[pl.ds(0, _KROWS)], ebuf.at[b], sem_l[b]).wait()
            pltpu.make_async_copy(
                id_hbm.at[pl.ds(0, _KROWS)], idbuf.at[b], sem_l[b]).wait()

        start_loads(0, 0)
        start_loads(1, 1)
        start_loads(2, 2)

        def drain_chunk(b):
            for j in range(_KROWS):
                pltpu.make_async_copy(
                    ebuf.at[b, j], sums_sh.at[idbuf.at[b, j]],
                    sem_s[b]).wait()
                pltpu.make_async_copy(
                    ones, cnts_sh.at[idbuf.at[b, j]], sem_s[b]).wait()

        def triple(k, carry):
            for b in range(3):
                i = k * 3 + b
                bp = (b + 2) % 3
                wait_loads(b)
                for j in range(_KROWS):
                    pltpu.async_copy(
                        ebuf.at[b, j], sums_sh.at[idbuf.at[b, j]], sem_s[b],
                        add=True)
                    pltpu.async_copy(
                        ones, cnts_sh.at[idbuf.at[b, j]], sem_s[b], add=True)

                @pl.when(i >= 1)
                def _():
                    drain_chunk(bp)

                @pl.when(i + 2 < nchunks)
                def _():
                    start_loads(i + 2, bp)
            return carry

        lax.fori_loop(0, nchunks // 3, triple, 0)
        drain_chunk((nchunks - 1) % 3)
        plsc.subcore_barrier()

        sl = pl.ds(s * _STRIPE, _STRIPE)
        pltpu.sync_copy(sums_sh.at[sl], psum_hbm.at[c, sl])
        pltpu.sync_copy(cnts_sh.at[sl], pcnt_hbm.at[c, sl])

    return sc_k


def _tc_finalize(ps_ref, pc_ref, o_ref):
    total = ps_ref[0] + ps_ref[1]
    count = pc_ref[0] + pc_ref[1]
    o_ref[...] = total / jnp.maximum(count, 1.0)


def kernel(site_energy, segment_ids, num_crystals):
    n = site_energy.shape[0]
    flat = site_energy.reshape(n)

    block = _NW * _KROWS * _LANE * 3
    n_pad = ((n + block - 1) // block) * block
    pad = n_pad - n
    flat = jnp.pad(flat, (0, pad))
    ids = jnp.pad(segment_ids, (0, pad), constant_values=_NUM_SEGMENTS)

    rows = n_pad // _LANE
    rows_per_worker = rows // _NW
    nchunks = rows_per_worker // _KROWS

    e2d = flat.reshape(rows, _LANE)
    id2d = ids.reshape(rows, _LANE)

    psum, pcnt = _make_sc_accumulate(rows_per_worker, nchunks)(e2d, id2d)

    srows = _SP // _LANE
    mean2d = pl.pallas_call(
        _tc_finalize,
        out_shape=jax.ShapeDtypeStruct((srows, _LANE), jnp.float32),
    )(psum.reshape(_NC, srows, _LANE), pcnt.reshape(_NC, srows, _LANE))

    return mean2d.reshape(_SP)[:_NUM_SEGMENTS, None]


# speedup vs baseline: 2.2181x; 1.0085x over previous
import functools

import jax
import jax.numpy as jnp
from jax import lax
from jax.experimental import pallas as pl
from jax.experimental.pallas import tpu as pltpu
from jax.experimental.pallas import tpu_sc as plsc

_NUM_SEGMENTS = 100000
_LANE = 128
_NC = 2
_NS = 16
_NW = _NC * _NS
_KROWS = 8

_SP = ((_NUM_SEGMENTS + _NS * _LANE - 1) // (_NS * _LANE)) * (_NS * _LANE)
_STRIPE = _SP // _NS


def _make_sc_accumulate(rows_per_worker, nchunks):
    mesh = plsc.VectorSubcoreMesh(core_axis_name="c", subcore_axis_name="s")

    @functools.partial(
        pl.kernel,
        mesh=mesh,
        out_type=(
            jax.ShapeDtypeStruct((_NC, _SP), jnp.float32),
            jax.ShapeDtypeStruct((_NC, _SP), jnp.float32),
        ),
        scratch_types=[
            pltpu.VMEM((3, _KROWS, _LANE), jnp.float32),
            pltpu.VMEM((3, _KROWS, _LANE), jnp.int32),
            pltpu.VMEM((_LANE,), jnp.float32),
            pltpu.VMEM((_STRIPE,), jnp.float32),
            pltpu.VMEM_SHARED((_SP,), jnp.float32),
            pltpu.VMEM_SHARED((_SP,), jnp.float32),
            pltpu.SemaphoreType.DMA,
            pltpu.SemaphoreType.DMA,
            pltpu.SemaphoreType.DMA,
            pltpu.SemaphoreType.DMA,
            pltpu.SemaphoreType.DMA,
            pltpu.SemaphoreType.DMA,
        ],
    )
    def sc_k(e_hbm, id_hbm, psum_hbm, pcnt_hbm,
             ebuf, idbuf, ones, zbuf, sums_sh, cnts_sh,
             sem_l0, sem_l1, sem_l2, sem_s0, sem_s1, sem_s2):
        c = lax.axis_index("c")
        s = lax.axis_index("s")
        wid = s * _NC + c

        zero16 = jnp.zeros((16,), jnp.float32)
        for i in range(_LANE // 16):
            ones[pl.ds(i * 16, 16)] = zero16 + 1.0

        def zfill(i, carry):
            zbuf[pl.ds(i * 16, 16)] = zero16
            return carry

        lax.fori_loop(0, _STRIPE // 16, zfill, 0)
        pltpu.sync_copy(zbuf, sums_sh.at[pl.ds(s * _STRIPE, _STRIPE)])
        pltpu.sync_copy(zbuf, cnts_sh.at[pl.ds(s * _STRIPE, _STRIPE)])
        plsc.subcore_barrier()

        base = wid * rows_per_worker
        sem_l = (sem_l0, sem_l1, sem_l2)
        sem_s = (sem_s0, sem_s1, sem_s2)

        def start_loads(i, b):
            r0 = base + i * _KROWS
            pltpu.async_copy(e_hbm.at[pl.ds(r0, _KROWS)], ebuf.at[b], sem_l[b])
            pltpu.async_copy(id_hbm.at[pl.ds(r0, _KROWS)], idbuf.at[b], sem_l[b])

        def wait_loads(b):
            pltpu.make_async_copy(
                e_hbm.at[pl.ds(0, _KROWS)], ebuf.at[b], sem_l[b]).wait()
            pltpu.make_async_copy(
                id_hbm.at[pl.ds(0, _KROWS)], idbuf.at[b], sem_l[b]).wait()

        start_loads(0, 0)
        start_loads(1, 1)
        start_loads(2, 2)

        def drain_chunk(b):
            for j in range(_KROWS):
                pltpu.make_async_copy(
                    ebuf.at[b, j], sums_sh.at[idbuf.at[b, j]],
                    sem_s[b]).wait()
                pltpu.make_async_copy(
                    ones, cnts_sh.at[idbuf.at[b, j]], sem_s[b]).wait()

        def triple(k, carry):
            for b in range(3):
                i = k * 3 + b
                bp = (b + 2) % 3
                wait_loads(b)
                for j in range(_KROWS):
                    pltpu.async_copy(
                        ebuf.at[b, j], sums_sh.at[idbuf.at[b, j]], sem_s[b],
                        add=True)
                    pltpu.async_copy(
                        ones, cnts_sh.at[idbuf.at[b, j]], sem_s[b], add=True)

                @pl.when(i >= 1)
                def _():
                    drain_chunk(bp)

                @pl.when(jnp.logical_and(i >= 1, i + 2 < nchunks))
                def _():
                    start_loads(i + 2, bp)
            return carry

        lax.fori_loop(0, nchunks // 3, triple, 0)
        drain_chunk((nchunks - 1) % 3)
        plsc.subcore_barrier()

        sl = pl.ds(s * _STRIPE, _STRIPE)
        pltpu.sync_copy(sums_sh.at[sl], psum_hbm.at[c, sl])
        pltpu.sync_copy(cnts_sh.at[sl], pcnt_hbm.at[c, sl])

    return sc_k


def _tc_finalize(ps_ref, pc_ref, o_ref):
    total = ps_ref[0] + ps_ref[1]
    count = pc_ref[0] + pc_ref[1]
    o_ref[...] = total / jnp.maximum(count, 1.0)


def kernel(site_energy, segment_ids, num_crystals):
    n = site_energy.shape[0]
    flat = site_energy.reshape(n)

    block = _NW * _KROWS * _LANE * 3
    n_pad = ((n + block - 1) // block) * block
    pad = n_pad - n
    flat = jnp.pad(flat, (0, pad))
    ids = jnp.pad(segment_ids, (0, pad), constant_values=_NUM_SEGMENTS)

    rows = n_pad // _LANE
    rows_per_worker = rows // _NW
    nchunks = rows_per_worker // _KROWS

    e2d = flat.reshape(rows, _LANE)
    id2d = ids.reshape(rows, _LANE)

    psum, pcnt = _make_sc_accumulate(rows_per_worker, nchunks)(e2d, id2d)

    srows = _SP // _LANE
    mean2d = pl.pallas_call(
        _tc_finalize,
        out_shape=jax.ShapeDtypeStruct((srows, _LANE), jnp.float32),
    )(psum.reshape(_NC, srows, _LANE), pcnt.reshape(_NC, srows, _LANE))

    return mean2d.reshape(_SP)[:_NUM_SEGMENTS, None]


# speedup vs baseline: 2.2225x; 1.0020x over previous
import functools

import jax
import jax.numpy as jnp
from jax import lax
from jax.experimental import pallas as pl
from jax.experimental.pallas import tpu as pltpu
from jax.experimental.pallas import tpu_sc as plsc

_NUM_SEGMENTS = 100000
_LANE = 128
_NC = 2
_NS = 16
_NW = _NC * _NS
_KROWS = 16

_SP = ((_NUM_SEGMENTS + _NS * _LANE - 1) // (_NS * _LANE)) * (_NS * _LANE)
_STRIPE = _SP // _NS


def _make_sc_accumulate(rows_per_worker, nchunks):
    mesh = plsc.VectorSubcoreMesh(core_axis_name="c", subcore_axis_name="s")

    @functools.partial(
        pl.kernel,
        mesh=mesh,
        out_type=(
            jax.ShapeDtypeStruct((_NC, _SP), jnp.float32),
            jax.ShapeDtypeStruct((_NC, _SP), jnp.float32),
        ),
        scratch_types=[
            pltpu.VMEM((3, _KROWS, _LANE), jnp.float32),
            pltpu.VMEM((3, _KROWS, _LANE), jnp.int32),
            pltpu.VMEM((_LANE,), jnp.float32),
            pltpu.VMEM((_STRIPE,), jnp.float32),
            pltpu.VMEM_SHARED((_SP,), jnp.float32),
            pltpu.VMEM_SHARED((_SP,), jnp.float32),
            pltpu.SemaphoreType.DMA,
            pltpu.SemaphoreType.DMA,
            pltpu.SemaphoreType.DMA,
            pltpu.SemaphoreType.DMA,
            pltpu.SemaphoreType.DMA,
            pltpu.SemaphoreType.DMA,
        ],
    )
    def sc_k(e_hbm, id_hbm, psum_hbm, pcnt_hbm,
             ebuf, idbuf, ones, zbuf, sums_sh, cnts_sh,
             sem_l0, sem_l1, sem_l2, sem_s0, sem_s1, sem_s2):
        c = lax.axis_index("c")
        s = lax.axis_index("s")
        wid = s * _NC + c

        zero16 = jnp.zeros((16,), jnp.float32)
        for i in range(_LANE // 16):
            ones[pl.ds(i * 16, 16)] = zero16 + 1.0

        def zfill(i, carry):
            zbuf[pl.ds(i * 16, 16)] = zero16
            return carry

        lax.fori_loop(0, _STRIPE // 16, zfill, 0)
        pltpu.sync_copy(zbuf, sums_sh.at[pl.ds(s * _STRIPE, _STRIPE)])
        pltpu.sync_copy(zbuf, cnts_sh.at[pl.ds(s * _STRIPE, _STRIPE)])
        plsc.subcore_barrier()

        base = wid * rows_per_worker
        sem_l = (sem_l0, sem_l1, sem_l2)
        sem_s = (sem_s0, sem_s1, sem_s2)

        def start_loads(i, b):
            r0 = base + i * _KROWS
            pltpu.async_copy(e_hbm.at[pl.ds(r0, _KROWS)], ebuf.at[b], sem_l[b])
            pltpu.async_copy(id_hbm.at[pl.ds(r0, _KROWS)], idbuf.at[b], sem_l[b])

        def wait_loads(b):
            pltpu.make_async_copy(
                e_hbm.at[pl.ds(0, _KROWS)], ebuf.at[b], sem_l[b]).wait()
            pltpu.make_async_copy(
                id_hbm.at[pl.ds(0, _KROWS)], idbuf.at[b], sem_l[b]).wait()

        start_loads(0, 0)
        start_loads(1, 1)
        start_loads(2, 2)

        def drain_chunk(b):
            for j in range(_KROWS):
                pltpu.make_async_copy(
                    ebuf.at[b, j], sums_sh.at[idbuf.at[b, j]],
                    sem_s[b]).wait()
                pltpu.make_async_copy(
                    ones, cnts_sh.at[idbuf.at[b, j]], sem_s[b]).wait()

        def triple(k, carry):
            for b in range(3):
                i = k * 3 + b
                bp = (b + 2) % 3
                wait_loads(b)
                for j in range(_KROWS):
                    pltpu.async_copy(
                        ebuf.at[b, j], sums_sh.at[idbuf.at[b, j]], sem_s[b],
                        add=True)
                    pltpu.async_copy(
                        ones, cnts_sh.at[idbuf.at[b, j]], sem_s[b], add=True)

                @pl.when(i >= 1)
                def _():
                    drain_chunk(bp)

                @pl.when(jnp.logical_and(i >= 1, i + 2 < nchunks))
                def _():
                    start_loads(i + 2, bp)
            return carry

        lax.fori_loop(0, nchunks // 3, triple, 0)
        drain_chunk((nchunks - 1) % 3)
        plsc.subcore_barrier()

        sl = pl.ds(s * _STRIPE, _STRIPE)
        pltpu.sync_copy(sums_sh.at[sl], psum_hbm.at[c, sl])
        pltpu.sync_copy(cnts_sh.at[sl], pcnt_hbm.at[c, sl])

    return sc_k


def _tc_finalize(ps_ref, pc_ref, o_ref):
    total = ps_ref[0] + ps_ref[1]
    count = pc_ref[0] + pc_ref[1]
    o_ref[...] = total / jnp.maximum(count, 1.0)


def kernel(site_energy, segment_ids, num_crystals):
    n = site_energy.shape[0]
    flat = site_energy.reshape(n)

    block = _NW * _KROWS * _LANE * 3
    n_pad = ((n + block - 1) // block) * block
    pad = n_pad - n
    flat = jnp.pad(flat, (0, pad))
    ids = jnp.pad(segment_ids, (0, pad), constant_values=_NUM_SEGMENTS)

    rows = n_pad // _LANE
    rows_per_worker = rows // _NW
    nchunks = rows_per_worker // _KROWS

    e2d = flat.reshape(rows, _LANE)
    id2d = ids.reshape(rows, _LANE)

    psum, pcnt = _make_sc_accumulate(rows_per_worker, nchunks)(e2d, id2d)

    srows = _SP // _LANE
    mean2d = pl.pallas_call(
        _tc_finalize,
        out_shape=jax.ShapeDtypeStruct((srows, _LANE), jnp.float32),
    )(psum.reshape(_NC, srows, _LANE), pcnt.reshape(_NC, srows, _LANE))

    return mean2d.reshape(_SP)[:_NUM_SEGMENTS, None]
